# baseline (speedup 1.0000x reference)
import functools

import jax
import jax.numpy as jnp
from jax.experimental import pallas as pl
from jax.experimental.pallas import tpu as pltpu

_BB = 128


def _crf_block_kernel(feats_ref, trans_ref, score_ref, path_ref, psi_ref):
    BB, T, S = feats_ref.shape
    trans = trans_ref[...]

    jidx3 = jax.lax.broadcasted_iota(jnp.int32, (BB, S, S), 2)

    def fwd(t, delta):
        feat_t = feats_ref[:, t, :]
        scores = trans[None, :, :] + delta[:, None, :]
        m = jnp.max(scores, axis=-1)
        eq = scores == m[:, :, None]
        psi = jnp.min(jnp.where(eq, jidx3, S), axis=-1)
        psi_ref[t] = psi
        return m + feat_t

    delta0 = jnp.full((BB, S), -10000.0, dtype=jnp.float32)
    final_delta = jax.lax.fori_loop(1, T, fwd, delta0, unroll=2)

    score_ref[...] = jnp.max(final_delta, axis=-1, keepdims=True)

    jidx2 = jax.lax.broadcasted_iota(jnp.int32, (BB, S), 1)
    m2 = jnp.max(final_delta, axis=-1, keepdims=True)
    last_tag = jnp.min(
        jnp.where(final_delta == m2, jidx2, S), axis=-1, keepdims=True
    )

    tidx = jax.lax.broadcasted_iota(jnp.int32, (BB, T), 1)
    acc0 = jnp.where(tidx == T - 1, last_tag, 0)

    def bwd(k, carry):
        tag, acc = carry
        t = T - 1 - k
        psi_t = psi_ref[t]
        eq = jidx2 == tag
        cur = jnp.sum(jnp.where(eq, psi_t, 0), axis=-1, keepdims=True)
        acc = jnp.where(tidx == t - 1, cur, acc)
        return cur, acc

    _, path = jax.lax.fori_loop(0, T - 1, bwd, (last_tag, acc0), unroll=2)
    path_ref[...] = path


def kernel(feats, transitions):
    B, T, S = feats.shape
    bb = _BB
    grid = (B // bb,)

    score, path = pl.pallas_call(
        _crf_block_kernel,
        grid=grid,
        in_specs=[
            pl.BlockSpec((bb, T, S), lambda b: (b, 0, 0)),
            pl.BlockSpec((S, S), lambda b: (0, 0)),
        ],
        out_specs=[
            pl.BlockSpec((bb, 1), lambda b: (b, 0)),
            pl.BlockSpec((bb, T), lambda b: (b, 0)),
        ],
        out_shape=[
            jax.ShapeDtypeStruct((B, 1), jnp.float32),
            jax.ShapeDtypeStruct((B, T), jnp.int32),
        ],
        scratch_shapes=[pltpu.VMEM((T, bb, S), jnp.int32)],
        compiler_params=pltpu.CompilerParams(
            dimension_semantics=("arbitrary",),
        ),
    )(feats, transitions)

    return score.reshape(B), path


# speedup vs baseline: 11.1736x; 11.1736x over previous
import jax
import jax.numpy as jnp
from jax.experimental import pallas as pl
from jax.experimental.pallas import tpu as pltpu

_BB = 128


def _crf_block_kernel(featsT_ref, transB_ref, score_ref, path_ref, psi_ref):
    T, S, BB = featsT_ref.shape
    jidx = jax.lax.broadcasted_iota(jnp.int32, (S, S, BB), 1).astype(jnp.float32)
    sidx = jax.lax.broadcasted_iota(jnp.int32, (S, BB), 0).astype(jnp.float32)

    def fwd(t, delta):
        feat_t = featsT_ref[t]
        scores = transB_ref[...] + delta[None, :, :]
        m = jnp.max(scores, axis=1)
        eq = scores == m[:, None, :]
        psi = jnp.min(jnp.where(eq, jidx, float(S)), axis=1)
        psi_ref[t] = psi
        return m + feat_t

    delta0 = jnp.full((S, BB), -10000.0, dtype=jnp.float32)
    final_delta = jax.lax.fori_loop(1, T, fwd, delta0)

    m2 = jnp.max(final_delta, axis=0, keepdims=True)
    score_ref[0] = m2
    last_tag = jnp.min(
        jnp.where(final_delta == m2, sidx, float(S)), axis=0, keepdims=True
    )
    path_ref[T - 1] = last_tag.astype(jnp.int32)

    def bwd(k, tag):
        t = T - 1 - k
        psi_t = psi_ref[t]
        eq = sidx == tag
        cur = jnp.max(jnp.where(eq, psi_t, -1.0), axis=0, keepdims=True)
        path_ref[t - 1] = cur.astype(jnp.int32)
        return cur

    jax.lax.fori_loop(0, T - 1, bwd, last_tag)


def kernel(feats, transitions):
    B, T, S = feats.shape
    bb = _BB
    grid = (B // bb,)

    featsT = jnp.transpose(feats, (1, 2, 0))
    transB = jnp.broadcast_to(transitions[:, :, None], (S, S, bb))

    score, pathT = pl.pallas_call(
        _crf_block_kernel,
        grid=grid,
        in_specs=[
            pl.BlockSpec((T, S, bb), lambda b: (0, 0, b)),
            pl.BlockSpec((S, S, bb), lambda b: (0, 0, 0)),
        ],
        out_specs=[
            pl.BlockSpec((1, 1, bb), lambda b: (0, 0, b)),
            pl.BlockSpec((T, 1, bb), lambda b: (0, 0, b)),
        ],
        out_shape=[
            jax.ShapeDtypeStruct((1, 1, B), jnp.float32),
            jax.ShapeDtypeStruct((T, 1, B), jnp.int32),
        ],
        scratch_shapes=[pltpu.VMEM((T, S, bb), jnp.float32)],
        compiler_params=pltpu.CompilerParams(
            dimension_semantics=("arbitrary",),
        ),
    )(featsT, transB)

    return score.reshape(B), pathT.reshape(T, B).T


# speedup vs baseline: 11.4161x; 1.0217x over previous
import jax
import jax.numpy as jnp
from jax.experimental import pallas as pl
from jax.experimental.pallas import tpu as pltpu

_BB = 128


def _crf_block_kernel(featsT_ref, transB_ref, score_ref, path_ref, psi_ref):
    T, S, BB = featsT_ref.shape
    sidx = jax.lax.broadcasted_iota(jnp.int32, (S, BB), 0).astype(jnp.float32)
    jidx = sidx[None, :, :]

    def fwd(t, delta):
        feat_t = featsT_ref[t]
        scores = transB_ref[...] + delta[None, :, :]
        m = jnp.max(scores, axis=1)
        eq = scores == m[:, None, :]
        psi = jnp.min(jnp.where(eq, jidx, float(S)), axis=1)
        psi_ref[t] = psi
        return m + feat_t

    delta0 = jnp.full((S, BB), -10000.0, dtype=jnp.float32)
    final_delta = jax.lax.fori_loop(1, T, fwd, delta0, unroll=2)

    m2 = jnp.max(final_delta, axis=0, keepdims=True)
    score_ref[0] = m2
    last_tag = jnp.min(
        jnp.where(final_delta == m2, sidx, float(S)), axis=0, keepdims=True
    )
    path_ref[T - 1] = last_tag.astype(jnp.int32)

    def bwd(k, tag):
        t = T - 1 - k
        psi_t = psi_ref[t]
        eq = sidx == tag
        cur = jnp.max(jnp.where(eq, psi_t, -1.0), axis=0, keepdims=True)
        path_ref[t - 1] = cur.astype(jnp.int32)
        return cur

    jax.lax.fori_loop(0, T - 1, bwd, last_tag)


def kernel(feats, transitions):
    B, T, S = feats.shape
    bb = _BB
    grid = (B // bb,)

    featsT = jnp.transpose(feats, (1, 2, 0))
    transB = jnp.broadcast_to(transitions[:, :, None], (S, S, bb))

    score, pathT = pl.pallas_call(
        _crf_block_kernel,
        grid=grid,
        in_specs=[
            pl.BlockSpec((T, S, bb), lambda b: (0, 0, b)),
            pl.BlockSpec((S, S, bb), lambda b: (0, 0, 0)),
        ],
        out_specs=[
            pl.BlockSpec((1, 1, bb), lambda b: (0, 0, b)),
            pl.BlockSpec((T, 1, bb), lambda b: (0, 0, b)),
        ],
        out_shape=[
            jax.ShapeDtypeStruct((1, 1, B), jnp.float32),
            jax.ShapeDtypeStruct((T, 1, B), jnp.int32),
        ],
        scratch_shapes=[pltpu.VMEM((T, S, bb), jnp.float32)],
        compiler_params=pltpu.CompilerParams(
            dimension_semantics=("arbitrary",),
        ),
    )(featsT, transB)

    return score.reshape(B), pathT.reshape(T, B).T
